# Initial kernel scaffold; baseline (speedup 1.0000x reference)
import jax
import jax.numpy as jnp
from jax.experimental import pallas as pl


def kernel(inputs, labels, gamma, beta, bn_gamma, bn_beta):
    raise NotImplementedError("write your pallas kernel here")


# speedup vs baseline: 1.1058x; 1.1058x over previous
import functools

import jax
import jax.numpy as jnp
from jax import lax
from jax.experimental import pallas as pl
from jax.experimental.pallas import tpu as pltpu
from jax.experimental.pallas import tpu_sc as plsc

B, H, W, C = 32, 56, 56, 384
HW = H * W
N = B * HW
EPS = 1e-3


_ROWS_PER_WORKER = 8
_NUM_WORKERS = B // _ROWS_PER_WORKER


@functools.cache
def _make_sc_gather():
    @functools.partial(
        pl.kernel,
        out_type=[
            jax.ShapeDtypeStruct((B, C), jnp.float32),
            jax.ShapeDtypeStruct((B, C), jnp.float32),
        ],
        mesh=plsc.VectorSubcoreMesh(core_axis_name="c", subcore_axis_name="s"),
        scratch_types=[
            pltpu.VMEM((_ROWS_PER_WORKER,), jnp.int32),
            pltpu.VMEM((_ROWS_PER_WORKER, C), jnp.float32),
            pltpu.VMEM((_ROWS_PER_WORKER, C), jnp.float32),
            pltpu.SemaphoreType.DMA,
        ],
    )
    def _sc_gather(labels_hbm, gamma_hbm, beta_hbm, g_out, b_out,
                   idx_v, rows_g, rows_b, sem):
        wid = lax.axis_index("s") * 2 + lax.axis_index("c")

        @pl.when(wid < _NUM_WORKERS)
        def _():
            base = wid * _ROWS_PER_WORKER
            pltpu.sync_copy(labels_hbm.at[pl.ds(base, _ROWS_PER_WORKER)], idx_v)
            pltpu.async_copy(gamma_hbm.at[idx_v], rows_g, sem).wait()
            pltpu.sync_copy(rows_g, g_out.at[pl.ds(base, _ROWS_PER_WORKER)])
            pltpu.async_copy(beta_hbm.at[idx_v], rows_b, sem).wait()
            pltpu.sync_copy(rows_b, b_out.at[pl.ds(base, _ROWS_PER_WORKER)])

    return _sc_gather


def _reduce_body(x_ref, out_ref):
    i = pl.program_id(0)
    x = x_ref[0]
    s = jnp.sum(x, axis=0, keepdims=True)
    ss = jnp.sum(x * x, axis=0, keepdims=True)

    @pl.when(i == 0)
    def _():
        out_ref[0:1, :] = s
        out_ref[1:2, :] = ss

    @pl.when(i > 0)
    def _():
        out_ref[0:1, :] += s
        out_ref[1:2, :] += ss


def _apply_body(stats_ref, bng_ref, bnb_ref, g_ref, b_ref, x_ref, o_ref):
    inv_n = jnp.float32(1.0 / N)
    mu = stats_ref[0:1, :] * inv_n
    var = stats_ref[1:2, :] * inv_n - mu * mu
    rstd = lax.rsqrt(var + jnp.float32(EPS))
    a = rstd * bng_ref[0:1, :]
    g = g_ref[0]
    scale = a * g
    shift = (bnb_ref[0:1, :] - mu * a) * g + b_ref[0]
    o_ref[0] = x_ref[0] * scale + shift


def kernel(inputs, labels, gamma, beta, bn_gamma, bn_beta):
    x3 = inputs.reshape(B, HW, C)
    labels_i = labels.astype(jnp.int32)

    g_rows, b_rows = _make_sc_gather()(labels_i, gamma, beta)

    stats = pl.pallas_call(
        _reduce_body,
        grid=(B,),
        in_specs=[pl.BlockSpec((1, HW, C), lambda i: (i, 0, 0))],
        out_specs=pl.BlockSpec((2, C), lambda i: (0, 0)),
        out_shape=jax.ShapeDtypeStruct((2, C), jnp.float32),
    )(x3)

    g3 = g_rows.reshape(B, 1, C)
    b3 = b_rows.reshape(B, 1, C)
    out = pl.pallas_call(
        _apply_body,
        grid=(B,),
        in_specs=[
            pl.BlockSpec((2, C), lambda i: (0, 0)),
            pl.BlockSpec((1, C), lambda i: (0, 0)),
            pl.BlockSpec((1, C), lambda i: (0, 0)),
            pl.BlockSpec((1, 1, C), lambda i: (i, 0, 0)),
            pl.BlockSpec((1, 1, C), lambda i: (i, 0, 0)),
            pl.BlockSpec((1, HW, C), lambda i: (i, 0, 0)),
        ],
        out_specs=pl.BlockSpec((1, HW, C), lambda i: (i, 0, 0)),
        out_shape=jax.ShapeDtypeStruct((B, HW, C), jnp.float32),
    )(stats, bn_gamma.reshape(1, C), bn_beta.reshape(1, C), g3, b3, x3)

    return out.reshape(B, H, W, C)
